# baseline (speedup 1.0000x reference)
import functools

import jax
import jax.numpy as jnp
from jax import lax
from jax.experimental import pallas as pl
from jax.experimental.pallas import tpu as pltpu
from jax.experimental.pallas import tpu_sc as plsc

_B, _W = 16384, 100
_NC, _NS, _L = 2, 16, 16
_NW = _NC * _NS
_RPW = _B // _NW
_GROUPS = _RPW // _L


def _make_sc_call():
    mesh = plsc.VectorSubcoreMesh(
        core_axis_name="c", subcore_axis_name="s",
        num_cores=_NC, num_subcores=_NS)

    @functools.partial(
        pl.kernel,
        mesh=mesh,
        compiler_params=pltpu.CompilerParams(needs_layout_passes=False),
        out_type=jax.ShapeDtypeStruct((_NW, _L), jnp.float32),
        scratch_types=[
            pltpu.VMEM((_RPW * _W,), jnp.float32),
            pltpu.VMEM((_RPW * _W,), jnp.float32),
            pltpu.VMEM((_W * _W,), jnp.float32),
            pltpu.VMEM((_L,), jnp.float32),
        ],
    )
    def sc_loss(predict_hbm, target_hbm, pm_hbm, out_hbm,
                pred_v, targ_v, pm_v, acc_v):
        wid = lax.axis_index("s") * _NC + lax.axis_index("c")
        base = wid * (_RPW * _W)
        pltpu.sync_copy(predict_hbm.at[pl.ds(base, _RPW * _W)], pred_v)
        pltpu.sync_copy(target_hbm.at[pl.ds(base, _RPW * _W)], targ_v)
        pltpu.sync_copy(pm_hbm, pm_v)

        lanes = lax.iota(jnp.int32, _L)
        zero_f = jnp.zeros((_L,), jnp.float32)
        zero_i = jnp.zeros((_L,), jnp.int32)

        def group_body(g, acc):
            rowoff = (g * _L + lanes) * _W

            def pass_max(j, carry):
                m, a = carry
                p = plsc.load_gather(pred_v, [rowoff + j])
                upd = p > m
                jv = jnp.full((_L,), j, jnp.int32)
                return jnp.where(upd, p, m), jnp.where(upd, jv, a)

            m0 = plsc.load_gather(pred_v, [rowoff])
            m, a = lax.fori_loop(1, _W, pass_max, (m0, zero_i))

            def pass_expsum(j, s):
                p = plsc.load_gather(pred_v, [rowoff + j])
                return s + jnp.exp(p - m)

            s = lax.fori_loop(0, _W, pass_expsum, zero_f)

            def pass_targ(j, carry):
                tm, t = carry
                p = plsc.load_gather(targ_v, [rowoff + j])
                upd = p > tm
                jv = jnp.full((_L,), j, jnp.int32)
                return jnp.where(upd, p, tm), jnp.where(upd, jv, t)

            t0 = plsc.load_gather(targ_v, [rowoff])
            _, t = lax.fori_loop(1, _W, pass_targ, (t0, zero_i))

            pm_val = plsc.load_gather(pm_v, [t * _W + a])
            contrib = jnp.where(a != t, pm_val / s, zero_f)
            return acc + contrib

        acc = lax.fori_loop(0, _GROUPS, group_body, zero_f)
        acc_v[...] = acc
        pltpu.sync_copy(acc_v, out_hbm.at[wid])

    return sc_loss


_SC_LOSS_CACHE = []


def kernel(predict, target, penalty_matrix):
    if not _SC_LOSS_CACHE:
        _SC_LOSS_CACHE.append(jax.jit(_make_sc_call()))
    partials = _SC_LOSS_CACHE[0](
        predict.reshape(-1), target.reshape(-1), penalty_matrix.reshape(-1))
    return jnp.sum(partials) / jnp.float32(predict.shape[0])


# speedup vs baseline: 1.4609x; 1.4609x over previous
import functools

import jax
import jax.numpy as jnp
from jax import lax
from jax.experimental import pallas as pl
from jax.experimental.pallas import tpu as pltpu
from jax.experimental.pallas import tpu_sc as plsc

_B, _W = 16384, 100
_NC, _NS, _L = 2, 16, 16
_NW = _NC * _NS
_RPW = _B // _NW
_GROUPS = _RPW // _L


def _make_sc_call():
    mesh = plsc.VectorSubcoreMesh(
        core_axis_name="c", subcore_axis_name="s",
        num_cores=_NC, num_subcores=_NS)

    @functools.partial(
        pl.kernel,
        mesh=mesh,
        compiler_params=pltpu.CompilerParams(needs_layout_passes=False),
        out_type=jax.ShapeDtypeStruct((_NW, _L), jnp.float32),
        scratch_types=[
            pltpu.VMEM((_RPW * _W,), jnp.float32),
            pltpu.VMEM((_RPW * _W,), jnp.float32),
            pltpu.VMEM((_W * _W,), jnp.float32),
            pltpu.VMEM((_L,), jnp.float32),
        ],
    )
    def sc_loss(predict_hbm, target_hbm, pm_hbm, out_hbm,
                pred_v, targ_v, pm_v, acc_v):
        wid = lax.axis_index("s") * _NC + lax.axis_index("c")
        base = wid * (_RPW * _W)
        pltpu.sync_copy(predict_hbm.at[pl.ds(base, _RPW * _W)], pred_v)
        pltpu.sync_copy(target_hbm.at[pl.ds(base, _RPW * _W)], targ_v)
        pltpu.sync_copy(pm_hbm, pm_v)

        lanes = lax.iota(jnp.int32, _L)
        zero_f = jnp.zeros((_L,), jnp.float32)
        zero_i = jnp.zeros((_L,), jnp.int32)

        neg_inf = jnp.full((_L,), -jnp.inf, jnp.float32)

        def group_body(g, acc):
            rowoff = (g * _L + lanes) * _W

            m, a, s = neg_inf, zero_i, zero_f
            tm, t = neg_inf, zero_i
            for j in range(_W):
                p = plsc.load_gather(pred_v, [rowoff + j])
                s = s + jnp.exp(p)
                upd = p > m
                m = jnp.where(upd, p, m)
                a = jnp.where(upd, jnp.full((_L,), j, jnp.int32), a)
                q = plsc.load_gather(targ_v, [rowoff + j])
                upd2 = q > tm
                tm = jnp.where(upd2, q, tm)
                t = jnp.where(upd2, jnp.full((_L,), j, jnp.int32), t)

            pm_val = plsc.load_gather(pm_v, [t * _W + a])
            contrib = jnp.where(a != t, pm_val * jnp.exp(m) / s, zero_f)
            return acc + contrib

        acc = lax.fori_loop(0, _GROUPS, group_body, zero_f)
        acc_v[...] = acc
        pltpu.sync_copy(acc_v, out_hbm.at[wid])

    return sc_loss


_SC_LOSS_CACHE = []


def kernel(predict, target, penalty_matrix):
    if not _SC_LOSS_CACHE:
        _SC_LOSS_CACHE.append(jax.jit(_make_sc_call()))
    partials = _SC_LOSS_CACHE[0](
        predict.reshape(-1), target.reshape(-1), penalty_matrix.reshape(-1))
    return jnp.sum(partials) / jnp.float32(predict.shape[0])
